# baseline (speedup 1.0000x reference)
import jax
import jax.numpy as jnp
from jax.experimental import pallas as pl

SIZE = 512
NCH = 20
NCHP = 24
HPROF = 30
HPROFP = 32
CINP = NCHP + HPROFP
HID = 256
SSN = 3
KW = 5
PAD = KW // 2
SPB = 8


def _shift_cols(v, s, pos):
    if s == 0:
        return v
    rolled = jnp.roll(v, -s, axis=1)
    valid = ((pos + s >= 0) & (pos + s < SIZE)).astype(v.dtype)
    return rolled * valid


def _dp_kernel(x_ref, seq_ref, ss_ref, w1_ref, b1_ref, w2_ref, b2_ref,
               out_ref):
    spb = x_ref.shape[0]
    x = jnp.concatenate([x_ref[j] for j in range(spb)], axis=1)
    m = (jnp.sum(x, axis=0, keepdims=True) > 0.0).astype(jnp.float32)
    seqt = jnp.concatenate([seq_ref[...]] * spb, axis=1) * m
    zx = jnp.zeros((NCHP - NCH, x.shape[1]), jnp.float32)
    zs = jnp.zeros((HPROFP - HPROF, x.shape[1]), jnp.float32)
    hmm = jnp.concatenate([x, zx, seqt, zs], axis=0)

    lanes = hmm.shape[1]
    pos = jax.lax.broadcasted_iota(jnp.int32, (1, lanes), 1) & (SIZE - 1)
    cols = jnp.concatenate(
        [_shift_cols(hmm, k - PAD, pos) for k in range(KW)], axis=0)

    rr = jax.lax.broadcasted_iota(jnp.int32, (KW * (NCH + HPROF), KW * CINP), 0)
    cc = jax.lax.broadcasted_iota(jnp.int32, (KW * (NCH + HPROF), KW * CINP), 1)
    k_of = cc // CINP
    j_of = cc % CINP
    i_of = jnp.where(j_of < NCH, j_of, j_of - (NCHP - NCH))
    valid = (j_of < NCH) | ((j_of >= NCHP) & (j_of < NCHP + HPROF))
    perm = ((rr == i_of * KW + k_of) & valid).astype(jnp.float32)
    w1r = jnp.dot(w1_ref[...], perm, preferred_element_type=jnp.float32)

    h = jnp.dot(w1r.astype(jnp.bfloat16), cols.astype(jnp.bfloat16),
                preferred_element_type=jnp.float32)
    h = jnp.maximum(h + b1_ref[...], 0.0)

    acc = jnp.dot(w2_ref[...], h, preferred_element_type=jnp.float32)
    lg = _shift_cols(acc[0:8], -PAD, pos)
    for k in range(1, KW):
        lg = lg + _shift_cols(acc[8 * k:8 * k + 8], k - PAD, pos)
    logits = lg[0:SSN] + b2_ref[...]

    mx = jnp.max(logits, axis=0, keepdims=True)
    e = jnp.exp(logits - mx)
    inv_e = jax.lax.rsqrt(jnp.sum(e * e, axis=0, keepdims=True))
    ss = jnp.concatenate([ss_ref[j] for j in range(spb)], axis=1)
    ss_n = ss * jax.lax.rsqrt(jnp.sum(ss * ss, axis=0, keepdims=True))
    dp = jnp.sum(e * ss_n, axis=0, keepdims=True) * inv_e * m
    for j in range(spb):
        out_ref[pl.ds(j, 1), :] = dp[:, j * SIZE:(j + 1) * SIZE]


def kernel(x, Q, SEQ_HMM, SS_HMM, W1, b1, W2, b2):
    del Q
    B = x.shape[0]
    spb = SPB if B % SPB == 0 else 1
    G = B // spb
    w1f = W1.reshape(HID, (NCH + HPROF) * KW)
    w2r = jnp.transpose(W2, (2, 0, 1))
    w2r = jnp.pad(w2r, ((0, 0), (0, 8 - SSN), (0, 0))).reshape(KW * 8, HID)
    return pl.pallas_call(
        _dp_kernel,
        grid=(G,),
        in_specs=[
            pl.BlockSpec((spb, NCH, SIZE), lambda n: (n, 0, 0)),
            pl.BlockSpec((HPROF, SIZE), lambda n: (0, 0)),
            pl.BlockSpec((spb, SSN, SIZE), lambda n: (n, 0, 0)),
            pl.BlockSpec((HID, (NCH + HPROF) * KW), lambda n: (0, 0)),
            pl.BlockSpec((HID, 1), lambda n: (0, 0)),
            pl.BlockSpec((KW * 8, HID), lambda n: (0, 0)),
            pl.BlockSpec((SSN, 1), lambda n: (0, 0)),
        ],
        out_specs=pl.BlockSpec((spb, SIZE), lambda n: (n, 0)),
        out_shape=jax.ShapeDtypeStruct((B, SIZE), jnp.float32),
    )(x, SEQ_HMM, SS_HMM, w1f, b1.reshape(HID, 1), w2r, b2.reshape(SSN, 1))


# speedup vs baseline: 1.0008x; 1.0008x over previous
import jax
import jax.numpy as jnp
from jax.experimental import pallas as pl

SIZE = 512
NCH = 20
NCHP = 24
HPROF = 30
HPROFP = 32
CINP = NCHP + HPROFP
HID = 256
SSN = 3
KW = 5
PAD = KW // 2
SPB = 8


def _shift_cols(v, s, pos):
    if s == 0:
        return v
    rolled = jnp.roll(v, -s, axis=1)
    valid = ((pos + s >= 0) & (pos + s < SIZE)).astype(v.dtype)
    return rolled * valid


def _dp_kernel(x_ref, seq_ref, ss_ref, w1_ref, b1_ref, w2_ref, b2_ref,
               out_ref):
    spb = x_ref.shape[0]
    x = jnp.concatenate([x_ref[j] for j in range(spb)], axis=1)
    m = (jnp.sum(x, axis=0, keepdims=True) > 0.0).astype(jnp.float32)
    seqt = jnp.concatenate([seq_ref[...]] * spb, axis=1) * m
    zx = jnp.zeros((NCHP - NCH, x.shape[1]), jnp.float32)
    zs = jnp.zeros((HPROFP - HPROF, x.shape[1]), jnp.float32)
    hmm = jnp.concatenate([x, zx, seqt, zs], axis=0)

    lanes = hmm.shape[1]
    pos = jax.lax.broadcasted_iota(jnp.int32, (1, lanes), 1) & (SIZE - 1)
    cols = jnp.concatenate(
        [_shift_cols(hmm, k - PAD, pos) for k in range(KW)], axis=0)

    rr = jax.lax.broadcasted_iota(jnp.int32, (KW * (NCH + HPROF), KW * CINP), 0)
    cc = jax.lax.broadcasted_iota(jnp.int32, (KW * (NCH + HPROF), KW * CINP), 1)
    k_of = cc // CINP
    j_of = cc % CINP
    i_of = jnp.where(j_of < NCH, j_of, j_of - (NCHP - NCH))
    valid = (j_of < NCH) | ((j_of >= NCHP) & (j_of < NCHP + HPROF))
    perm = ((rr == i_of * KW + k_of) & valid).astype(jnp.float32)
    w1r = jnp.dot(w1_ref[...], perm, preferred_element_type=jnp.float32)

    h = jnp.dot(w1r, cols, preferred_element_type=jnp.float32)
    h = jnp.maximum(h + b1_ref[...], 0.0)

    zrow = jnp.zeros((8 - SSN, HID * KW), jnp.float32)
    w2rep = jnp.concatenate(
        [jnp.concatenate([w2_ref[...], zrow], axis=0)] * KW, axis=0)
    mr = jax.lax.broadcasted_iota(jnp.int32, (KW * 8, HID * KW), 0)
    mf = jax.lax.broadcasted_iota(jnp.int32, (KW * 8, HID * KW), 1)
    w2m = jnp.where(mf % KW == mr // 8, w2rep, 0.0)
    br = jax.lax.broadcasted_iota(jnp.int32, (HID * KW, HID), 0)
    bc = jax.lax.broadcasted_iota(jnp.int32, (HID * KW, HID), 1)
    bridge = (br // KW == bc).astype(jnp.float32)
    w2stack = jnp.dot(w2m, bridge, preferred_element_type=jnp.float32)

    acc = jnp.dot(w2stack, h, preferred_element_type=jnp.float32)
    lg = _shift_cols(acc[0:8], -PAD, pos)
    for k in range(1, KW):
        lg = lg + _shift_cols(acc[8 * k:8 * k + 8], k - PAD, pos)
    logits = lg[0:SSN] + b2_ref[...]

    mx = jnp.max(logits, axis=0, keepdims=True)
    e = jnp.exp(logits - mx)
    inv_e = jax.lax.rsqrt(jnp.sum(e * e, axis=0, keepdims=True))
    ss = jnp.concatenate([ss_ref[j] for j in range(spb)], axis=1)
    ss_n = ss * jax.lax.rsqrt(jnp.sum(ss * ss, axis=0, keepdims=True))
    dp = jnp.sum(e * ss_n, axis=0, keepdims=True) * inv_e * m
    for j in range(spb):
        out_ref[pl.ds(j, 1), :] = dp[:, j * SIZE:(j + 1) * SIZE]


def kernel(x, Q, SEQ_HMM, SS_HMM, W1, b1, W2, b2):
    del Q
    B = x.shape[0]
    spb = SPB if B % SPB == 0 else 1
    G = B // spb
    w1f = W1.reshape(HID, (NCH + HPROF) * KW)
    w2f = W2.reshape(SSN, HID * KW)
    return pl.pallas_call(
        _dp_kernel,
        grid=(G,),
        in_specs=[
            pl.BlockSpec((spb, NCH, SIZE), lambda n: (n, 0, 0)),
            pl.BlockSpec((HPROF, SIZE), lambda n: (0, 0)),
            pl.BlockSpec((spb, SSN, SIZE), lambda n: (n, 0, 0)),
            pl.BlockSpec((HID, (NCH + HPROF) * KW), lambda n: (0, 0)),
            pl.BlockSpec((HID, 1), lambda n: (0, 0)),
            pl.BlockSpec((SSN, HID * KW), lambda n: (0, 0)),
            pl.BlockSpec((SSN, 1), lambda n: (0, 0)),
        ],
        out_specs=pl.BlockSpec((spb, SIZE), lambda n: (n, 0)),
        out_shape=jax.ShapeDtypeStruct((B, SIZE), jnp.float32),
    )(x, SEQ_HMM, SS_HMM, w1f, b1.reshape(HID, 1), w2f, b2.reshape(SSN, 1))


# speedup vs baseline: 1.0363x; 1.0355x over previous
import jax
import jax.numpy as jnp
from jax.experimental import pallas as pl

SIZE = 512
NCH = 20
NCHP = 24
HPROF = 30
HPROFP = 32
CINP = NCHP + HPROFP
HID = 256
SSN = 3
KW = 5
PAD = KW // 2
SPB = 16


def _shift_cols(v, s, pos):
    if s == 0:
        return v
    rolled = jnp.roll(v, -s, axis=1)
    valid = ((pos + s >= 0) & (pos + s < SIZE)).astype(v.dtype)
    return rolled * valid


def _dp_kernel(x_ref, seq_ref, ss_ref, w1_ref, b1_ref, w2_ref, b2_ref,
               out_ref):
    spb = x_ref.shape[0]
    x = jnp.concatenate([x_ref[j] for j in range(spb)], axis=1)
    m = (jnp.sum(x, axis=0, keepdims=True) > 0.0).astype(jnp.float32)
    seqt = jnp.concatenate([seq_ref[...]] * spb, axis=1) * m
    zx = jnp.zeros((NCHP - NCH, x.shape[1]), jnp.float32)
    zs = jnp.zeros((HPROFP - HPROF, x.shape[1]), jnp.float32)
    hmm = jnp.concatenate([x, zx, seqt, zs], axis=0)

    lanes = hmm.shape[1]
    pos = jax.lax.broadcasted_iota(jnp.int32, (1, lanes), 1) & (SIZE - 1)
    cols = jnp.concatenate(
        [_shift_cols(hmm, k - PAD, pos) for k in range(KW)], axis=0)

    rr = jax.lax.broadcasted_iota(jnp.int32, (KW * (NCH + HPROF), KW * CINP), 0)
    cc = jax.lax.broadcasted_iota(jnp.int32, (KW * (NCH + HPROF), KW * CINP), 1)
    k_of = cc // CINP
    j_of = cc % CINP
    i_of = jnp.where(j_of < NCH, j_of, j_of - (NCHP - NCH))
    valid = (j_of < NCH) | ((j_of >= NCHP) & (j_of < NCHP + HPROF))
    perm = ((rr == i_of * KW + k_of) & valid).astype(jnp.float32)
    w1r = jnp.dot(w1_ref[...], perm, preferred_element_type=jnp.float32)

    h = jnp.dot(w1r, cols, preferred_element_type=jnp.float32)
    h = jnp.maximum(h + b1_ref[...], 0.0)

    zrow = jnp.zeros((8 - SSN, HID * KW), jnp.float32)
    w2rep = jnp.concatenate(
        [jnp.concatenate([w2_ref[...], zrow], axis=0)] * KW, axis=0)
    mr = jax.lax.broadcasted_iota(jnp.int32, (KW * 8, HID * KW), 0)
    mf = jax.lax.broadcasted_iota(jnp.int32, (KW * 8, HID * KW), 1)
    w2m = jnp.where(mf % KW == mr // 8, w2rep, 0.0)
    br = jax.lax.broadcasted_iota(jnp.int32, (HID * KW, HID), 0)
    bc = jax.lax.broadcasted_iota(jnp.int32, (HID * KW, HID), 1)
    bridge = (br // KW == bc).astype(jnp.float32)
    w2stack = jnp.dot(w2m, bridge, preferred_element_type=jnp.float32)

    acc = jnp.dot(w2stack, h, preferred_element_type=jnp.float32)
    lg = _shift_cols(acc[0:8], -PAD, pos)
    for k in range(1, KW):
        lg = lg + _shift_cols(acc[8 * k:8 * k + 8], k - PAD, pos)
    logits = lg[0:SSN] + b2_ref[...]

    mx = jnp.max(logits, axis=0, keepdims=True)
    e = jnp.exp(logits - mx)
    inv_e = jax.lax.rsqrt(jnp.sum(e * e, axis=0, keepdims=True))
    ss = jnp.concatenate([ss_ref[j] for j in range(spb)], axis=1)
    ss_n = ss * jax.lax.rsqrt(jnp.sum(ss * ss, axis=0, keepdims=True))
    dp = jnp.sum(e * ss_n, axis=0, keepdims=True) * inv_e * m
    for j in range(spb):
        out_ref[pl.ds(j, 1), :] = dp[:, j * SIZE:(j + 1) * SIZE]


def kernel(x, Q, SEQ_HMM, SS_HMM, W1, b1, W2, b2):
    del Q
    B = x.shape[0]
    spb = SPB if B % SPB == 0 else 1
    G = B // spb
    w1f = W1.reshape(HID, (NCH + HPROF) * KW)
    w2f = W2.reshape(SSN, HID * KW)
    return pl.pallas_call(
        _dp_kernel,
        grid=(G,),
        in_specs=[
            pl.BlockSpec((spb, NCH, SIZE), lambda n: (n, 0, 0)),
            pl.BlockSpec((HPROF, SIZE), lambda n: (0, 0)),
            pl.BlockSpec((spb, SSN, SIZE), lambda n: (n, 0, 0)),
            pl.BlockSpec((HID, (NCH + HPROF) * KW), lambda n: (0, 0)),
            pl.BlockSpec((HID, 1), lambda n: (0, 0)),
            pl.BlockSpec((SSN, HID * KW), lambda n: (0, 0)),
            pl.BlockSpec((SSN, 1), lambda n: (0, 0)),
        ],
        out_specs=pl.BlockSpec((spb, SIZE), lambda n: (n, 0)),
        out_shape=jax.ShapeDtypeStruct((B, SIZE), jnp.float32),
    )(x, SEQ_HMM, SS_HMM, w1f, b1.reshape(HID, 1), w2f, b2.reshape(SSN, 1))


# speedup vs baseline: 1.0366x; 1.0002x over previous
import jax
import jax.numpy as jnp
from jax.experimental import pallas as pl

SIZE = 512
NCH = 20
NCHP = 24
HPROF = 30
HPROFP = 32
CINP = NCHP + HPROFP
HID = 256
SSN = 3
KW = 5
PAD = KW // 2
SPB = 16


def _shift_cols(v, s, masks):
    if s == 0:
        return v
    return jnp.roll(v, -s, axis=1) * masks[s]


def _dp_kernel(x_ref, seq_ref, ss_ref, w1_ref, b1_ref, w2_ref, b2_ref,
               out_ref):
    spb = x_ref.shape[0]
    x = jnp.concatenate([x_ref[j] for j in range(spb)], axis=1)
    m = (jnp.sum(x, axis=0, keepdims=True) > 0.0).astype(jnp.float32)
    seqt = jnp.concatenate([seq_ref[...]] * spb, axis=1) * m
    zx = jnp.zeros((NCHP - NCH, x.shape[1]), jnp.float32)
    zs = jnp.zeros((HPROFP - HPROF, x.shape[1]), jnp.float32)
    hmm = jnp.concatenate([x, zx, seqt, zs], axis=0)

    lanes = hmm.shape[1]
    pos = jax.lax.broadcasted_iota(jnp.int32, (1, lanes), 1) & (SIZE - 1)
    masks = {s: ((pos + s >= 0) & (pos + s < SIZE)).astype(jnp.float32)
             for s in range(-PAD, PAD + 1) if s != 0}
    cols = jnp.concatenate(
        [_shift_cols(hmm, k - PAD, masks) for k in range(KW)], axis=0)

    rr = jax.lax.broadcasted_iota(jnp.int32, (KW * (NCH + HPROF), KW * CINP), 0)
    cc = jax.lax.broadcasted_iota(jnp.int32, (KW * (NCH + HPROF), KW * CINP), 1)
    k_of = cc // CINP
    j_of = cc % CINP
    i_of = jnp.where(j_of < NCH, j_of, j_of - (NCHP - NCH))
    valid = (j_of < NCH) | ((j_of >= NCHP) & (j_of < NCHP + HPROF))
    perm = ((rr == i_of * KW + k_of) & valid).astype(jnp.float32)
    w1r = jnp.dot(w1_ref[...], perm, preferred_element_type=jnp.float32)

    h = jnp.dot(w1r, cols, preferred_element_type=jnp.float32)
    h = jnp.maximum(h + b1_ref[...], 0.0)

    zrow = jnp.zeros((8 - SSN, HID * KW), jnp.float32)
    w2rep = jnp.concatenate(
        [jnp.concatenate([w2_ref[...], zrow], axis=0)] * KW, axis=0)
    mr = jax.lax.broadcasted_iota(jnp.int32, (KW * 8, HID * KW), 0)
    mf = jax.lax.broadcasted_iota(jnp.int32, (KW * 8, HID * KW), 1)
    w2m = jnp.where(mf % KW == mr // 8, w2rep, 0.0)
    br = jax.lax.broadcasted_iota(jnp.int32, (HID * KW, HID), 0)
    bc = jax.lax.broadcasted_iota(jnp.int32, (HID * KW, HID), 1)
    bridge = (br // KW == bc).astype(jnp.float32)
    w2stack = jnp.dot(w2m, bridge, preferred_element_type=jnp.float32)

    acc = jnp.dot(w2stack, h, preferred_element_type=jnp.float32)
    lg = _shift_cols(acc[0:8], -PAD, masks)
    for k in range(1, KW):
        lg = lg + _shift_cols(acc[8 * k:8 * k + 8], k - PAD, masks)
    logits = lg[0:SSN] + b2_ref[...]

    mx = jnp.max(logits, axis=0, keepdims=True)
    e = jnp.exp(logits - mx)
    inv_e = jax.lax.rsqrt(jnp.sum(e * e, axis=0, keepdims=True))
    ss = jnp.concatenate([ss_ref[j] for j in range(spb)], axis=1)
    ss_n = ss * jax.lax.rsqrt(jnp.sum(ss * ss, axis=0, keepdims=True))
    dp = jnp.sum(e * ss_n, axis=0, keepdims=True) * inv_e * m
    for j in range(spb):
        out_ref[pl.ds(j, 1), :] = dp[:, j * SIZE:(j + 1) * SIZE]


def kernel(x, Q, SEQ_HMM, SS_HMM, W1, b1, W2, b2):
    del Q
    B = x.shape[0]
    spb = SPB if B % SPB == 0 else 1
    G = B // spb
    w1f = W1.reshape(HID, (NCH + HPROF) * KW)
    w2f = W2.reshape(SSN, HID * KW)
    return pl.pallas_call(
        _dp_kernel,
        grid=(G,),
        in_specs=[
            pl.BlockSpec((spb, NCH, SIZE), lambda n: (n, 0, 0)),
            pl.BlockSpec((HPROF, SIZE), lambda n: (0, 0)),
            pl.BlockSpec((spb, SSN, SIZE), lambda n: (n, 0, 0)),
            pl.BlockSpec((HID, (NCH + HPROF) * KW), lambda n: (0, 0)),
            pl.BlockSpec((HID, 1), lambda n: (0, 0)),
            pl.BlockSpec((SSN, HID * KW), lambda n: (0, 0)),
            pl.BlockSpec((SSN, 1), lambda n: (0, 0)),
        ],
        out_specs=pl.BlockSpec((spb, SIZE), lambda n: (n, 0)),
        out_shape=jax.ShapeDtypeStruct((B, SIZE), jnp.float32),
    )(x, SEQ_HMM, SS_HMM, w1f, b1.reshape(HID, 1), w2f, b2.reshape(SSN, 1))
